# baseline (speedup 1.0000x reference)
import functools

import jax
import jax.numpy as jnp
from jax import lax
from jax.experimental import pallas as pl
from jax.experimental.pallas import tpu as pltpu
from jax.experimental.pallas import tpu_sc as plsc

N_NODES = 10000
N_PAD = 10240
D = 128
E = 320000
NC = 2
NS = 16
NW = NC * NS
CB = 96
NCH_C0 = 129
NCH_C1 = 81
NCHMAX = max(NCH_C0, NCH_C1)
EPAD = (NCH_C0 + NCH_C1) * NS * CB
NBUF = 3
IBUF = 6
GA = 2
SL = NBUF - GA
IP = 4
ACC_ROWS = 10240

_mesh = plsc.VectorSubcoreMesh(core_axis_name="c", subcore_axis_name="s")
_sc_params = pltpu.CompilerParams(needs_layout_passes=False)


@functools.partial(
    pl.kernel,
    mesh=_mesh,
    out_type=jax.ShapeDtypeStruct((NW, N_PAD), jnp.float32),
    scratch_types=[
        pltpu.VMEM((NCHMAX, 2, CB), jnp.int32),
        pltpu.VMEM((N_PAD,), jnp.float32),
    ],
    compiler_params=_sc_params,
)
def _sc_hist(eidx_hbm, out_hbm, idx_v, hist_v):
    c = lax.axis_index("c")
    s = lax.axis_index("s")
    wid = c * NS + s

    zero16 = jnp.zeros((16,), jnp.float32)

    def zbody(i, carry):
        hist_v[pl.ds(i * 16, 16)] = zero16
        return carry

    lax.fori_loop(0, N_PAD // 16, zbody, 0)

    pltpu.sync_copy(eidx_hbm.at[wid], idx_v)

    ones16 = jnp.ones((16,), jnp.float32)

    def body(ch, carry):
        def inner(j, carry2):
            idx = idx_v[ch, 1, pl.ds(j * 16, 16)]
            plsc.addupdate_scatter(hist_v, [idx], ones16)
            return carry2

        return lax.fori_loop(0, CB // 16, inner, carry)

    lax.fori_loop(0, NCHMAX, body, 0)

    pltpu.sync_copy(hist_v, out_hbm.at[wid])


ROWS_PER_TILE = ACC_ROWS // NS


@functools.partial(
    pl.kernel,
    mesh=_mesh,
    out_type=jax.ShapeDtypeStruct((NC, N_PAD, D), jnp.float32),
    scratch_types=[
        pltpu.VMEM((IBUF, 2, CB), jnp.int32),
        pltpu.VMEM((NBUF, CB, D), jnp.float32),
        pltpu.VMEM_SHARED((ACC_ROWS, D), jnp.float32),
        [pltpu.SemaphoreType.DMA] * IBUF,
        [pltpu.SemaphoreType.DMA] * NBUF,
        [pltpu.SemaphoreType.DMA] * NBUF,
    ],
    compiler_params=_sc_params,
)
def _sc_edge_agg(g_hbm, eidx_hbm, out_hbm,
                 iring, rows, acc, isem, gsem, ssem):
    c = lax.axis_index("c")
    s = lax.axis_index("s")
    wid = c * NS + s

    zero16 = jnp.zeros((16,), jnp.float32)

    def zb(i, carry):
        r = i // 8
        off = (i % 8) * 16
        rows[0, r, pl.ds(off, 16)] = zero16
        return carry

    lax.fori_loop(0, CB * 8, zb, 0)

    def zacc(j, carry):
        pltpu.sync_copy(rows.at[0],
                        acc.at[pl.ds(s * ROWS_PER_TILE + j * CB, CB)])
        return carry

    lax.fori_loop(0, ROWS_PER_TILE // CB, zacc, 0)
    if ROWS_PER_TILE % CB:
        pltpu.sync_copy(
            rows.at[0, pl.ds(0, ROWS_PER_TILE % CB)],
            acc.at[pl.ds(s * ROWS_PER_TILE + CB * (ROWS_PER_TILE // CB),
                         ROWS_PER_TILE % CB)])

    plsc.subcore_barrier()

    def i_copy(ch):
        j = ch % IBUF
        return pltpu.make_async_copy(eidx_hbm.at[wid, ch], iring.at[j],
                                     isem[j])

    def g_copy(ch, b, j):
        return pltpu.make_async_copy(g_hbm.at[iring.at[j, 0]], rows.at[b],
                                     gsem[b])

    def s_copy(b, j):
        return pltpu.make_async_copy(rows.at[b], acc.at[iring.at[j, 1]],
                                     ssem[b])

    def pipeline(nchunk):
        def step(ch):
            b = ch % NBUF
            j = ch % IBUF
            g_copy(ch, b, j).wait()
            s_copy(b, j).start(add=True)
            if ch >= SL:
                s_copy((ch - SL) % NBUF, (ch - SL) % IBUF).wait()
            if ch + IP < nchunk:
                i_copy(ch + IP).start()
            if ch + GA < nchunk:
                i_copy(ch + GA).wait()
                g_copy(ch + GA, (ch + GA) % NBUF, (ch + GA) % IBUF).start()

        for ch in range(IP):
            i_copy(ch).start()
        for ch in range(GA):
            i_copy(ch).wait()
            g_copy(ch, ch % NBUF, ch % IBUF).start()

        for ch in range(IBUF):
            step(ch)

        def group(i, carry):
            base = IBUF + i * IBUF
            for k in range(IBUF):
                ch = base + k
                b = k % NBUF
                j = k % IBUF
                pltpu.make_async_copy(g_hbm.at[iring.at[j, 0]], rows.at[b],
                                      gsem[b]).wait()
                s_copy(b, j).start(add=True)
                s_copy((k - SL) % NBUF, (k - SL) % IBUF).wait()
                nj = (k + IP) % IBUF
                pltpu.make_async_copy(eidx_hbm.at[wid, ch + IP], iring.at[nj],
                                      isem[nj]).start()
                nb = (k + GA) % NBUF
                mj = (k + GA) % IBUF
                pltpu.make_async_copy(eidx_hbm.at[wid, ch + GA], iring.at[mj],
                                      isem[mj]).wait()
                pltpu.make_async_copy(g_hbm.at[iring.at[mj, 0]], rows.at[nb],
                                      gsem[nb]).start()
            return carry

        n_groups = (nchunk - 2 * IBUF) // IBUF
        lax.fori_loop(0, n_groups, group, 0)

        for ch in range(IBUF + n_groups * IBUF, nchunk):
            step(ch)

        for ch in range(nchunk - SL, nchunk):
            s_copy(ch % NBUF, ch % IBUF).wait()

    @pl.when(c == 0)
    def _core0():
        pipeline(NCH_C0)

    @pl.when(c == 1)
    def _core1():
        pipeline(NCH_C1)

    plsc.subcore_barrier()

    pltpu.sync_copy(acc.at[pl.ds(s * ROWS_PER_TILE, ROWS_PER_TILE)],
                    out_hbm.at[c, pl.ds(s * ROWS_PER_TILE, ROWS_PER_TILE)])


BLK = 1280
GRID = N_PAD // BLK


def _dis(hist_blk):
    cnt = jnp.sum(hist_blk, axis=0) + 1.0
    return lax.rsqrt(cnt)[:, None]


def _tc1_body(x_ref, w_ref, hist_ref, g_ref):
    dis = _dis(hist_ref[...])
    h = jnp.dot(x_ref[...], w_ref[...], preferred_element_type=jnp.float32)
    g_ref[...] = h * dis


def _tc2_body(a0_ref, a1_ref, g_ref, hist_ref, b_ref, w_ref, out_ref):
    dis = _dis(hist_ref[...])
    t = (a0_ref[...] + a1_ref[...] + g_ref[...]) * dis
    h = jnp.maximum(t + b_ref[...], 0.0)
    out_ref[...] = jnp.dot(h, w_ref[...],
                           preferred_element_type=jnp.float32) * dis


def _tc3_body(a0_ref, a1_ref, g_ref, hist_ref, b_ref, out_ref):
    dis = _dis(hist_ref[...])
    out_ref[...] = (a0_ref[...] + a1_ref[...] + g_ref[...]) * dis + b_ref[...]


_row_spec = pl.BlockSpec((BLK, D), lambda i: (i, 0))
_mat_spec = pl.BlockSpec((D, D), lambda i: (0, 0))
_hist_spec = pl.BlockSpec((NW, BLK), lambda i: (0, i))
_bias_spec = pl.BlockSpec((1, D), lambda i: (0, 0))
_out_rows = jax.ShapeDtypeStruct((N_PAD, D), jnp.float32)

_tc1 = pl.pallas_call(
    _tc1_body,
    grid=(GRID,),
    in_specs=[_row_spec, _mat_spec, _hist_spec],
    out_specs=_row_spec,
    out_shape=_out_rows,
)

_tc2 = pl.pallas_call(
    _tc2_body,
    grid=(GRID,),
    in_specs=[_row_spec, _row_spec, _row_spec, _hist_spec, _bias_spec,
              _mat_spec],
    out_specs=_row_spec,
    out_shape=_out_rows,
)

_tc3 = pl.pallas_call(
    _tc3_body,
    grid=(GRID,),
    in_specs=[_row_spec, _row_spec, _row_spec, _hist_spec, _bias_spec],
    out_specs=_row_spec,
    out_shape=_out_rows,
)


@jax.jit
def kernel(x, edge_index, W1, b1, W2, b2):
    src = edge_index[0].astype(jnp.int32)
    dst = edge_index[1].astype(jnp.int32)

    npad_e = EPAD - E
    src_p = jnp.concatenate([src, jnp.zeros((npad_e,), jnp.int32)])
    dst_p = jnp.concatenate([dst, jnp.full((npad_e,), N_NODES, jnp.int32)])

    n0 = NS * NCH_C0 * CB
    cpad0 = ((0, 0), (0, NCHMAX - NCH_C0), (0, 0), (0, 0))
    cpad1 = ((0, 0), (0, NCHMAX - NCH_C1), (0, 0), (0, 0))
    s0 = jnp.pad(src_p[:n0].reshape(NS, NCH_C0, 1, CB), cpad0)
    d0 = jnp.pad(dst_p[:n0].reshape(NS, NCH_C0, 1, CB), cpad0,
                 constant_values=N_NODES)
    s1 = jnp.pad(src_p[n0:].reshape(NS, NCH_C1, 1, CB), cpad1)
    d1 = jnp.pad(dst_p[n0:].reshape(NS, NCH_C1, 1, CB), cpad1,
                 constant_values=N_NODES)
    eidx = jnp.concatenate(
        [jnp.concatenate([s0, d0], axis=2),
         jnp.concatenate([s1, d1], axis=2)], axis=0)

    x_p = jnp.zeros((N_PAD, D), x.dtype).at[:N_NODES].set(x)
    b1r = b1.reshape(1, D)
    b2r = b2.reshape(1, D)

    hists = _sc_hist(eidx)

    g1 = _tc1(x_p, W1, hists)
    acc1 = _sc_edge_agg(g1, eidx)
    g2 = _tc2(acc1[0], acc1[1], g1, hists, b1r, W2)
    acc2 = _sc_edge_agg(g2, eidx)
    out = _tc3(acc2[0], acc2[1], g2, hists, b2r)
    return out[:N_NODES]


# speedup vs baseline: 1.0106x; 1.0106x over previous
import functools

import jax
import jax.numpy as jnp
from jax import lax
from jax.experimental import pallas as pl
from jax.experimental.pallas import tpu as pltpu
from jax.experimental.pallas import tpu_sc as plsc

N_NODES = 10000
N_PAD = 10240
D = 128
E = 320000
NC = 2
NS = 16
NW = NC * NS
CB = 96
NCH_C0 = 136
NCH_C1 = 74
NCHMAX = max(NCH_C0, NCH_C1)
EPAD = (NCH_C0 + NCH_C1) * NS * CB
NBUF = 3
IBUF = 6
GA = 2
SL = NBUF - GA
IP = 4
ACC_ROWS = 10240

_mesh = plsc.VectorSubcoreMesh(core_axis_name="c", subcore_axis_name="s")
_sc_params = pltpu.CompilerParams(needs_layout_passes=False)


@functools.partial(
    pl.kernel,
    mesh=_mesh,
    out_type=jax.ShapeDtypeStruct((NW, N_PAD), jnp.float32),
    scratch_types=[
        pltpu.VMEM((NCHMAX, 2, CB), jnp.int32),
        pltpu.VMEM((N_PAD,), jnp.float32),
    ],
    compiler_params=_sc_params,
)
def _sc_hist(eidx_hbm, out_hbm, idx_v, hist_v):
    c = lax.axis_index("c")
    s = lax.axis_index("s")
    wid = c * NS + s

    zero16 = jnp.zeros((16,), jnp.float32)

    def zbody(i, carry):
        hist_v[pl.ds(i * 16, 16)] = zero16
        return carry

    lax.fori_loop(0, N_PAD // 16, zbody, 0)

    pltpu.sync_copy(eidx_hbm.at[wid], idx_v)

    ones16 = jnp.ones((16,), jnp.float32)

    def body(ch, carry):
        def inner(j, carry2):
            idx = idx_v[ch, 1, pl.ds(j * 16, 16)]
            plsc.addupdate_scatter(hist_v, [idx], ones16)
            return carry2

        return lax.fori_loop(0, CB // 16, inner, carry)

    lax.fori_loop(0, NCHMAX, body, 0)

    pltpu.sync_copy(hist_v, out_hbm.at[wid])


ROWS_PER_TILE = ACC_ROWS // NS


@functools.partial(
    pl.kernel,
    mesh=_mesh,
    out_type=jax.ShapeDtypeStruct((NC, N_PAD, D), jnp.float32),
    scratch_types=[
        pltpu.VMEM((IBUF, 2, CB), jnp.int32),
        pltpu.VMEM((NBUF, CB, D), jnp.float32),
        pltpu.VMEM_SHARED((ACC_ROWS, D), jnp.float32),
        [pltpu.SemaphoreType.DMA] * IBUF,
        [pltpu.SemaphoreType.DMA] * NBUF,
        [pltpu.SemaphoreType.DMA] * NBUF,
    ],
    compiler_params=_sc_params,
)
def _sc_edge_agg(g_hbm, eidx_hbm, out_hbm,
                 iring, rows, acc, isem, gsem, ssem):
    c = lax.axis_index("c")
    s = lax.axis_index("s")
    wid = c * NS + s

    zero16 = jnp.zeros((16,), jnp.float32)

    def zb(i, carry):
        r = i // 8
        off = (i % 8) * 16
        rows[0, r, pl.ds(off, 16)] = zero16
        return carry

    lax.fori_loop(0, CB * 8, zb, 0)

    def zacc(j, carry):
        pltpu.sync_copy(rows.at[0],
                        acc.at[pl.ds(s * ROWS_PER_TILE + j * CB, CB)])
        return carry

    lax.fori_loop(0, ROWS_PER_TILE // CB, zacc, 0)
    if ROWS_PER_TILE % CB:
        pltpu.sync_copy(
            rows.at[0, pl.ds(0, ROWS_PER_TILE % CB)],
            acc.at[pl.ds(s * ROWS_PER_TILE + CB * (ROWS_PER_TILE // CB),
                         ROWS_PER_TILE % CB)])

    plsc.subcore_barrier()

    def i_copy(ch):
        j = ch % IBUF
        return pltpu.make_async_copy(eidx_hbm.at[wid, ch], iring.at[j],
                                     isem[j])

    def g_copy(ch, b, j):
        return pltpu.make_async_copy(g_hbm.at[iring.at[j, 0]], rows.at[b],
                                     gsem[b])

    def s_copy(b, j):
        return pltpu.make_async_copy(rows.at[b], acc.at[iring.at[j, 1]],
                                     ssem[b])

    def pipeline(nchunk):
        def step(ch):
            b = ch % NBUF
            j = ch % IBUF
            g_copy(ch, b, j).wait()
            s_copy(b, j).start(add=True)
            if ch >= SL:
                s_copy((ch - SL) % NBUF, (ch - SL) % IBUF).wait()
            if ch + IP < nchunk:
                i_copy(ch + IP).start()
            if ch + GA < nchunk:
                i_copy(ch + GA).wait()
                g_copy(ch + GA, (ch + GA) % NBUF, (ch + GA) % IBUF).start()

        for ch in range(IP):
            i_copy(ch).start()
        for ch in range(GA):
            i_copy(ch).wait()
            g_copy(ch, ch % NBUF, ch % IBUF).start()

        for ch in range(IBUF):
            step(ch)

        def group(i, carry):
            base = IBUF + i * IBUF
            for k in range(IBUF):
                ch = base + k
                b = k % NBUF
                j = k % IBUF
                pltpu.make_async_copy(g_hbm.at[iring.at[j, 0]], rows.at[b],
                                      gsem[b]).wait()
                s_copy(b, j).start(add=True)
                s_copy((k - SL) % NBUF, (k - SL) % IBUF).wait()
                nj = (k + IP) % IBUF
                pltpu.make_async_copy(eidx_hbm.at[wid, ch + IP], iring.at[nj],
                                      isem[nj]).start()
                nb = (k + GA) % NBUF
                mj = (k + GA) % IBUF
                pltpu.make_async_copy(eidx_hbm.at[wid, ch + GA], iring.at[mj],
                                      isem[mj]).wait()
                pltpu.make_async_copy(g_hbm.at[iring.at[mj, 0]], rows.at[nb],
                                      gsem[nb]).start()
            return carry

        n_groups = (nchunk - 2 * IBUF) // IBUF
        lax.fori_loop(0, n_groups, group, 0)

        for ch in range(IBUF + n_groups * IBUF, nchunk):
            step(ch)

        for ch in range(nchunk - SL, nchunk):
            s_copy(ch % NBUF, ch % IBUF).wait()

    @pl.when(c == 0)
    def _core0():
        pipeline(NCH_C0)

    @pl.when(c == 1)
    def _core1():
        pipeline(NCH_C1)

    plsc.subcore_barrier()

    pltpu.sync_copy(acc.at[pl.ds(s * ROWS_PER_TILE, ROWS_PER_TILE)],
                    out_hbm.at[c, pl.ds(s * ROWS_PER_TILE, ROWS_PER_TILE)])


BLK = 1280
GRID = N_PAD // BLK


def _dis(hist_blk):
    cnt = jnp.sum(hist_blk, axis=0) + 1.0
    return lax.rsqrt(cnt)[:, None]


def _tc1_body(x_ref, w_ref, hist_ref, g_ref):
    dis = _dis(hist_ref[...])
    h = jnp.dot(x_ref[...], w_ref[...], preferred_element_type=jnp.float32)
    g_ref[...] = h * dis


def _tc2_body(a0_ref, a1_ref, g_ref, hist_ref, b_ref, w_ref, out_ref):
    dis = _dis(hist_ref[...])
    t = (a0_ref[...] + a1_ref[...] + g_ref[...]) * dis
    h = jnp.maximum(t + b_ref[...], 0.0)
    out_ref[...] = jnp.dot(h, w_ref[...],
                           preferred_element_type=jnp.float32) * dis


def _tc3_body(a0_ref, a1_ref, g_ref, hist_ref, b_ref, out_ref):
    dis = _dis(hist_ref[...])
    out_ref[...] = (a0_ref[...] + a1_ref[...] + g_ref[...]) * dis + b_ref[...]


_row_spec = pl.BlockSpec((BLK, D), lambda i: (i, 0))
_mat_spec = pl.BlockSpec((D, D), lambda i: (0, 0))
_hist_spec = pl.BlockSpec((NW, BLK), lambda i: (0, i))
_bias_spec = pl.BlockSpec((1, D), lambda i: (0, 0))
_out_rows = jax.ShapeDtypeStruct((N_PAD, D), jnp.float32)

_tc1 = pl.pallas_call(
    _tc1_body,
    grid=(GRID,),
    in_specs=[_row_spec, _mat_spec, _hist_spec],
    out_specs=_row_spec,
    out_shape=_out_rows,
)

_tc2 = pl.pallas_call(
    _tc2_body,
    grid=(GRID,),
    in_specs=[_row_spec, _row_spec, _row_spec, _hist_spec, _bias_spec,
              _mat_spec],
    out_specs=_row_spec,
    out_shape=_out_rows,
)

_tc3 = pl.pallas_call(
    _tc3_body,
    grid=(GRID,),
    in_specs=[_row_spec, _row_spec, _row_spec, _hist_spec, _bias_spec],
    out_specs=_row_spec,
    out_shape=_out_rows,
)


@jax.jit
def kernel(x, edge_index, W1, b1, W2, b2):
    src = edge_index[0].astype(jnp.int32)
    dst = edge_index[1].astype(jnp.int32)

    npad_e = EPAD - E
    src_p = jnp.concatenate([src, jnp.zeros((npad_e,), jnp.int32)])
    dst_p = jnp.concatenate([dst, jnp.full((npad_e,), N_NODES, jnp.int32)])

    n0 = NS * NCH_C0 * CB
    cpad0 = ((0, 0), (0, NCHMAX - NCH_C0), (0, 0), (0, 0))
    cpad1 = ((0, 0), (0, NCHMAX - NCH_C1), (0, 0), (0, 0))
    s0 = jnp.pad(src_p[:n0].reshape(NS, NCH_C0, 1, CB), cpad0)
    d0 = jnp.pad(dst_p[:n0].reshape(NS, NCH_C0, 1, CB), cpad0,
                 constant_values=N_NODES)
    s1 = jnp.pad(src_p[n0:].reshape(NS, NCH_C1, 1, CB), cpad1)
    d1 = jnp.pad(dst_p[n0:].reshape(NS, NCH_C1, 1, CB), cpad1,
                 constant_values=N_NODES)
    eidx = jnp.concatenate(
        [jnp.concatenate([s0, d0], axis=2),
         jnp.concatenate([s1, d1], axis=2)], axis=0)

    x_p = jnp.zeros((N_PAD, D), x.dtype).at[:N_NODES].set(x)
    b1r = b1.reshape(1, D)
    b2r = b2.reshape(1, D)

    hists = _sc_hist(eidx)

    g1 = _tc1(x_p, W1, hists)
    acc1 = _sc_edge_agg(g1, eidx)
    g2 = _tc2(acc1[0], acc1[1], g1, hists, b1r, W2)
    acc2 = _sc_edge_agg(g2, eidx)
    out = _tc3(acc2[0], acc2[1], g2, hists, b2r)
    return out[:N_NODES]


# speedup vs baseline: 1.0194x; 1.0086x over previous
import functools

import jax
import jax.numpy as jnp
from jax import lax
from jax.experimental import pallas as pl
from jax.experimental.pallas import tpu as pltpu
from jax.experimental.pallas import tpu_sc as plsc

N_NODES = 10000
N_PAD = 10240
D = 128
E = 320000
NC = 2
NS = 16
NW = NC * NS
CB = 96
NCH_C0 = 150
NCH_C1 = 60
NCHMAX = max(NCH_C0, NCH_C1)
EPAD = (NCH_C0 + NCH_C1) * NS * CB
NBUF = 3
IBUF = 6
GA = 2
SL = NBUF - GA
IP = 4
ACC_ROWS = 10240

_mesh = plsc.VectorSubcoreMesh(core_axis_name="c", subcore_axis_name="s")
_sc_params = pltpu.CompilerParams(needs_layout_passes=False)


@functools.partial(
    pl.kernel,
    mesh=_mesh,
    out_type=jax.ShapeDtypeStruct((NW, N_PAD), jnp.float32),
    scratch_types=[
        pltpu.VMEM((NCHMAX, 2, CB), jnp.int32),
        pltpu.VMEM((N_PAD,), jnp.float32),
    ],
    compiler_params=_sc_params,
)
def _sc_hist(eidx_hbm, out_hbm, idx_v, hist_v):
    c = lax.axis_index("c")
    s = lax.axis_index("s")
    wid = c * NS + s

    zero16 = jnp.zeros((16,), jnp.float32)

    def zbody(i, carry):
        hist_v[pl.ds(i * 16, 16)] = zero16
        return carry

    lax.fori_loop(0, N_PAD // 16, zbody, 0)

    pltpu.sync_copy(eidx_hbm.at[wid], idx_v)

    ones16 = jnp.ones((16,), jnp.float32)

    def body(ch, carry):
        def inner(j, carry2):
            idx = idx_v[ch, 1, pl.ds(j * 16, 16)]
            plsc.addupdate_scatter(hist_v, [idx], ones16)
            return carry2

        return lax.fori_loop(0, CB // 16, inner, carry)

    lax.fori_loop(0, NCHMAX, body, 0)

    pltpu.sync_copy(hist_v, out_hbm.at[wid])


ROWS_PER_TILE = ACC_ROWS // NS


@functools.partial(
    pl.kernel,
    mesh=_mesh,
    out_type=jax.ShapeDtypeStruct((NC, N_PAD, D), jnp.float32),
    scratch_types=[
        pltpu.VMEM((IBUF, 2, CB), jnp.int32),
        pltpu.VMEM((NBUF, CB, D), jnp.float32),
        pltpu.VMEM_SHARED((ACC_ROWS, D), jnp.float32),
        [pltpu.SemaphoreType.DMA] * IBUF,
        [pltpu.SemaphoreType.DMA] * NBUF,
        [pltpu.SemaphoreType.DMA] * NBUF,
    ],
    compiler_params=_sc_params,
)
def _sc_edge_agg(g_hbm, eidx_hbm, out_hbm,
                 iring, rows, acc, isem, gsem, ssem):
    c = lax.axis_index("c")
    s = lax.axis_index("s")
    wid = c * NS + s

    zero16 = jnp.zeros((16,), jnp.float32)

    def zb(i, carry):
        r = i // 8
        off = (i % 8) * 16
        rows[0, r, pl.ds(off, 16)] = zero16
        return carry

    lax.fori_loop(0, CB * 8, zb, 0)

    def zacc(j, carry):
        pltpu.sync_copy(rows.at[0],
                        acc.at[pl.ds(s * ROWS_PER_TILE + j * CB, CB)])
        return carry

    lax.fori_loop(0, ROWS_PER_TILE // CB, zacc, 0)
    if ROWS_PER_TILE % CB:
        pltpu.sync_copy(
            rows.at[0, pl.ds(0, ROWS_PER_TILE % CB)],
            acc.at[pl.ds(s * ROWS_PER_TILE + CB * (ROWS_PER_TILE // CB),
                         ROWS_PER_TILE % CB)])

    plsc.subcore_barrier()

    def i_copy(ch):
        j = ch % IBUF
        return pltpu.make_async_copy(eidx_hbm.at[wid, ch], iring.at[j],
                                     isem[j])

    def g_copy(ch, b, j):
        return pltpu.make_async_copy(g_hbm.at[iring.at[j, 0]], rows.at[b],
                                     gsem[b])

    def s_copy(b, j):
        return pltpu.make_async_copy(rows.at[b], acc.at[iring.at[j, 1]],
                                     ssem[b])

    def pipeline(nchunk):
        def step(ch):
            b = ch % NBUF
            j = ch % IBUF
            g_copy(ch, b, j).wait()
            s_copy(b, j).start(add=True)
            if ch >= SL:
                s_copy((ch - SL) % NBUF, (ch - SL) % IBUF).wait()
            if ch + IP < nchunk:
                i_copy(ch + IP).start()
            if ch + GA < nchunk:
                i_copy(ch + GA).wait()
                g_copy(ch + GA, (ch + GA) % NBUF, (ch + GA) % IBUF).start()

        for ch in range(IP):
            i_copy(ch).start()
        for ch in range(GA):
            i_copy(ch).wait()
            g_copy(ch, ch % NBUF, ch % IBUF).start()

        for ch in range(IBUF):
            step(ch)

        def group(i, carry):
            base = IBUF + i * IBUF
            for k in range(IBUF):
                ch = base + k
                b = k % NBUF
                j = k % IBUF
                pltpu.make_async_copy(g_hbm.at[iring.at[j, 0]], rows.at[b],
                                      gsem[b]).wait()
                s_copy(b, j).start(add=True)
                s_copy((k - SL) % NBUF, (k - SL) % IBUF).wait()
                nj = (k + IP) % IBUF
                pltpu.make_async_copy(eidx_hbm.at[wid, ch + IP], iring.at[nj],
                                      isem[nj]).start()
                nb = (k + GA) % NBUF
                mj = (k + GA) % IBUF
                pltpu.make_async_copy(eidx_hbm.at[wid, ch + GA], iring.at[mj],
                                      isem[mj]).wait()
                pltpu.make_async_copy(g_hbm.at[iring.at[mj, 0]], rows.at[nb],
                                      gsem[nb]).start()
            return carry

        n_groups = (nchunk - 2 * IBUF) // IBUF
        lax.fori_loop(0, n_groups, group, 0)

        for ch in range(IBUF + n_groups * IBUF, nchunk):
            step(ch)

        for ch in range(nchunk - SL, nchunk):
            s_copy(ch % NBUF, ch % IBUF).wait()

    @pl.when(c == 0)
    def _core0():
        pipeline(NCH_C0)

    @pl.when(c == 1)
    def _core1():
        pipeline(NCH_C1)

    plsc.subcore_barrier()

    pltpu.sync_copy(acc.at[pl.ds(s * ROWS_PER_TILE, ROWS_PER_TILE)],
                    out_hbm.at[c, pl.ds(s * ROWS_PER_TILE, ROWS_PER_TILE)])


BLK = 1280
GRID = N_PAD // BLK


def _dis(hist_blk):
    cnt = jnp.sum(hist_blk, axis=0) + 1.0
    return lax.rsqrt(cnt)[:, None]


def _tc1_body(x_ref, w_ref, hist_ref, g_ref):
    dis = _dis(hist_ref[...])
    h = jnp.dot(x_ref[...], w_ref[...], preferred_element_type=jnp.float32)
    g_ref[...] = h * dis


def _tc2_body(a0_ref, a1_ref, g_ref, hist_ref, b_ref, w_ref, out_ref):
    dis = _dis(hist_ref[...])
    t = (a0_ref[...] + a1_ref[...] + g_ref[...]) * dis
    h = jnp.maximum(t + b_ref[...], 0.0)
    out_ref[...] = jnp.dot(h, w_ref[...],
                           preferred_element_type=jnp.float32) * dis


def _tc3_body(a0_ref, a1_ref, g_ref, hist_ref, b_ref, out_ref):
    dis = _dis(hist_ref[...])
    out_ref[...] = (a0_ref[...] + a1_ref[...] + g_ref[...]) * dis + b_ref[...]


_row_spec = pl.BlockSpec((BLK, D), lambda i: (i, 0))
_mat_spec = pl.BlockSpec((D, D), lambda i: (0, 0))
_hist_spec = pl.BlockSpec((NW, BLK), lambda i: (0, i))
_bias_spec = pl.BlockSpec((1, D), lambda i: (0, 0))
_out_rows = jax.ShapeDtypeStruct((N_PAD, D), jnp.float32)

_tc1 = pl.pallas_call(
    _tc1_body,
    grid=(GRID,),
    in_specs=[_row_spec, _mat_spec, _hist_spec],
    out_specs=_row_spec,
    out_shape=_out_rows,
)

_tc2 = pl.pallas_call(
    _tc2_body,
    grid=(GRID,),
    in_specs=[_row_spec, _row_spec, _row_spec, _hist_spec, _bias_spec,
              _mat_spec],
    out_specs=_row_spec,
    out_shape=_out_rows,
)

_tc3 = pl.pallas_call(
    _tc3_body,
    grid=(GRID,),
    in_specs=[_row_spec, _row_spec, _row_spec, _hist_spec, _bias_spec],
    out_specs=_row_spec,
    out_shape=_out_rows,
)


@jax.jit
def kernel(x, edge_index, W1, b1, W2, b2):
    src = edge_index[0].astype(jnp.int32)
    dst = edge_index[1].astype(jnp.int32)

    npad_e = EPAD - E
    src_p = jnp.concatenate([src, jnp.zeros((npad_e,), jnp.int32)])
    dst_p = jnp.concatenate([dst, jnp.full((npad_e,), N_NODES, jnp.int32)])

    n0 = NS * NCH_C0 * CB
    cpad0 = ((0, 0), (0, NCHMAX - NCH_C0), (0, 0), (0, 0))
    cpad1 = ((0, 0), (0, NCHMAX - NCH_C1), (0, 0), (0, 0))
    s0 = jnp.pad(src_p[:n0].reshape(NS, NCH_C0, 1, CB), cpad0)
    d0 = jnp.pad(dst_p[:n0].reshape(NS, NCH_C0, 1, CB), cpad0,
                 constant_values=N_NODES)
    s1 = jnp.pad(src_p[n0:].reshape(NS, NCH_C1, 1, CB), cpad1)
    d1 = jnp.pad(dst_p[n0:].reshape(NS, NCH_C1, 1, CB), cpad1,
                 constant_values=N_NODES)
    eidx = jnp.concatenate(
        [jnp.concatenate([s0, d0], axis=2),
         jnp.concatenate([s1, d1], axis=2)], axis=0)

    x_p = jnp.zeros((N_PAD, D), x.dtype).at[:N_NODES].set(x)
    b1r = b1.reshape(1, D)
    b2r = b2.reshape(1, D)

    hists = _sc_hist(eidx)

    g1 = _tc1(x_p, W1, hists)
    acc1 = _sc_edge_agg(g1, eidx)
    g2 = _tc2(acc1[0], acc1[1], g1, hists, b1r, W2)
    acc2 = _sc_edge_agg(g2, eidx)
    out = _tc3(acc2[0], acc2[1], g2, hists, b2r)
    return out[:N_NODES]
